# baseline (speedup 1.0000x reference)
import functools

import jax
import jax.numpy as jnp
from jax import lax
from jax.experimental import pallas as pl
from jax.experimental.pallas import tpu as pltpu
from jax.experimental.pallas import tpu_sc as plsc


def _make_gather(V, D, B, NC, NS, chunk):
    NW = NC * NS
    b_per_w = B // NW
    n_chunks = b_per_w // chunk
    mesh = plsc.VectorSubcoreMesh(core_axis_name="c", subcore_axis_name="s")

    @functools.partial(
        pl.kernel,
        mesh=mesh,
        out_type=jax.ShapeDtypeStruct((B, D), jnp.float32),
        compiler_params=pltpu.CompilerParams(use_tc_tiling_on_sc=False),
        scratch_types=[
            pltpu.VMEM((chunk,), jnp.int32),
            pltpu.VMEM((chunk, D), jnp.float32),
            pltpu.SemaphoreType.DMA,
        ],
    )
    def gather_kernel(table_hbm, idx_hbm, out_hbm, idx_v, rows_v, sem):
        wid = lax.axis_index("s") * NC + lax.axis_index("c")
        base = wid * b_per_w

        def body(g, carry):
            off = base + g * chunk
            pltpu.sync_copy(idx_hbm.at[pl.ds(off, chunk)], idx_v)
            pltpu.async_copy(table_hbm.at[idx_v], rows_v, sem).wait()
            pltpu.sync_copy(rows_v, out_hbm.at[pl.ds(off, chunk)])
            return carry

        lax.fori_loop(0, n_chunks, body, 0)

    return gather_kernel


def kernel(token_ids, W):
    S, T = token_ids.shape
    V, D = W.shape
    B = S * T
    idx_flat = token_ids.reshape(B).astype(jnp.int32)

    info = plsc.get_sparse_core_info()
    NC, NS = info.num_cores, info.num_subcores

    out = _make_gather(V, D, B, NC, NS, chunk=128)(W, idx_flat)
    return out.reshape(S, T, D)


# speedup vs baseline: 1.1959x; 1.1959x over previous
import functools

import jax
import jax.numpy as jnp
from jax import lax
from jax.experimental import pallas as pl
from jax.experimental.pallas import tpu as pltpu
from jax.experimental.pallas import tpu_sc as plsc

_CHUNK = 256
_NBUF = 4


def _make_gather(V, D, B, NC, NS):
    NW = NC * NS
    b_per_w = B // NW
    chunk = _CHUNK
    nbuf = _NBUF
    n_chunks = b_per_w // chunk
    n_outer = n_chunks // nbuf
    assert n_chunks % nbuf == 0 and b_per_w % chunk == 0
    mesh = plsc.VectorSubcoreMesh(core_axis_name="c", subcore_axis_name="s")

    @functools.partial(
        pl.kernel,
        mesh=mesh,
        out_type=jax.ShapeDtypeStruct((B, D), jnp.float32),
        compiler_params=pltpu.CompilerParams(use_tc_tiling_on_sc=False),
        scratch_types=[
            pltpu.VMEM((b_per_w,), jnp.int32),
            pltpu.VMEM((nbuf, chunk, D), jnp.float32),
            [pltpu.SemaphoreType.DMA] * _NBUF,
            [pltpu.SemaphoreType.DMA] * _NBUF,
        ],
    )
    def gather_kernel(table_hbm, idx_hbm, out_hbm, idx_v, rows_v, gsems, ssems):
        wid = lax.axis_index("s") * NC + lax.axis_index("c")
        base = wid * b_per_w
        pltpu.sync_copy(idx_hbm.at[pl.ds(base, b_per_w)], idx_v)

        def start_gather(g, b):
            pltpu.async_copy(
                table_hbm.at[idx_v.at[pl.ds(g * chunk, chunk)]],
                rows_v.at[b],
                gsems[b],
            )

        def wait_gather(b):
            pltpu.make_async_copy(
                table_hbm.at[idx_v.at[pl.ds(0, chunk)]], rows_v.at[b], gsems[b]
            ).wait()

        def start_store(g, b):
            pltpu.async_copy(
                rows_v.at[b],
                out_hbm.at[pl.ds(base + g * chunk, chunk)],
                ssems[b],
            )

        def wait_store(b):
            pltpu.make_async_copy(
                rows_v.at[b], out_hbm.at[pl.ds(base, chunk)], ssems[b]
            ).wait()

        for b in range(nbuf):
            start_gather(b, b)

        def outer(go, carry):
            for b in range(nbuf):
                g = go * nbuf + b
                wait_gather(b)
                start_store(g, b)
                wait_store(b)
                start_gather(g + nbuf, b)
            return carry

        lax.fori_loop(0, n_outer - 1, outer, 0)

        for b in range(nbuf):
            g = (n_outer - 1) * nbuf + b
            wait_gather(b)
            start_store(g, b)
        for b in range(nbuf):
            wait_store(b)

    return gather_kernel


def kernel(token_ids, W):
    S, T = token_ids.shape
    V, D = W.shape
    B = S * T
    idx_flat = token_ids.reshape(B).astype(jnp.int32)

    info = plsc.get_sparse_core_info()
    NC, NS = info.num_cores, info.num_subcores

    out = _make_gather(V, D, B, NC, NS)(W, idx_flat)
    return out.reshape(S, T, D)
